# baseline (speedup 1.0000x reference)
import functools

import jax
import jax.numpy as jnp
from jax import lax
from jax.experimental import pallas as pl
from jax.experimental.pallas import tpu as pltpu
from jax.experimental.pallas import tpu_sc as plsc

_NC, _NS = 2, 16
_NW = _NC * _NS
_D = 1024
_SEQ = 8192
_B = 4
_SC_B = 2
_T = 16
_ROWS_PER_W = _SEQ // _NW
_NCHUNK = _ROWS_PER_W // _T
_NSTEP = _NCHUNK * _SC_B


def _sc_body(x_hbm, pos_hbm, out_hbm,
             pos0, pos1, xb0, xb1, xb2, xb3,
             ps0, ps1, ls0, ls1, ls2, ls3, ss0, ss1, ss2, ss3):
    wid = lax.axis_index("s") * _NC + lax.axis_index("c")
    base = wid * _ROWS_PER_W
    posb = (pos0, pos1)
    xb = (xb0, xb1, xb2, xb3)
    ps = (ps0, ps1)
    ls = (ls0, ls1, ls2, ls3)
    ss = (ss0, ss1, ss2, ss3)

    nb = _SC_B

    def x_slice(g):
        return (g % nb, pl.ds(base + (g // nb) * _T, _T))

    def issue_load(g, j):
        b, rows = x_slice(g)
        pltpu.async_copy(x_hbm.at[(_B - nb) + b, rows], xb[j], ls[j])

    pltpu.async_copy(pos_hbm.at[pl.ds(base, _T)], pos0, ps0)
    issue_load(0, 0)
    issue_load(1, 1)

    @pl.loop(0, _NCHUNK, step=2)
    def _chunk_pair(cbase):
        for cc in range(2):
            c = cbase + cc
            pltpu.make_async_copy(
                pos_hbm.at[pl.ds(base + c * _T, _T)], posb[cc], ps[cc]
            ).wait()

            @pl.when(c + 1 < _NCHUNK)
            def _():
                pltpu.async_copy(
                    pos_hbm.at[pl.ds(base + (c + 1) * _T, _T)],
                    posb[1 - cc], ps[1 - cc])

            for b in range(nb):
                t = cc * nb + b
                g = cbase * nb + t
                j = t % 4
                jn = (t + 2) % 4

                @pl.when(g + 2 < _NSTEP)
                def _():
                    @pl.when(g >= 2)
                    def _():
                        bp, rp = x_slice(g - 2)
                        pltpu.make_async_copy(
                            xb[jn], out_hbm.at[bp, rp], ss[jn]).wait()
                    issue_load(g + 2, jn)

                bg, rg = x_slice(g)
                pltpu.make_async_copy(
                    x_hbm.at[(_B - nb) + bg, rg], xb[j], ls[j]).wait()

                @plsc.parallel_loop(0, _T * _D, step=16, unroll=8)
                def _add(k):
                    i = k >> 10
                    col = pl.multiple_of(k & (_D - 1), 16)
                    plsc.addupdate(xb[j].at[i, pl.ds(col, 16)],
                                   posb[cc][i, pl.ds(col, 16)])

                pltpu.async_copy(xb[j], out_hbm.at[bg, rg], ss[j])

    for t in range(4):
        g = _NSTEP - 4 + t
        bp, rp = x_slice(g)
        pltpu.make_async_copy(xb[g % 4], out_hbm.at[bp, rp], ss[g % 4]).wait()


_sc_kernel = functools.partial(
    pl.kernel,
    out_type=jax.ShapeDtypeStruct((_SC_B, _SEQ, _D), jnp.float32),
    mesh=plsc.VectorSubcoreMesh(
        core_axis_name="c", subcore_axis_name="s",
        num_cores=_NC, num_subcores=_NS,
    ),
    scratch_types=(
        [pltpu.VMEM((_T, _D), jnp.float32)] * 6
        + [pltpu.SemaphoreType.DMA] * 10
    ),
)(_sc_body)


_BS = 512


def _tc_body(x_ref, pos_ref, out_ref):
    out_ref[...] = x_ref[...] + pos_ref[...][None]


def _tc_kernel(x, position_embedding):
    n_blocks = _SEQ // _BS
    return pl.pallas_call(
        _tc_body,
        grid=(n_blocks, _B - _SC_B),
        in_specs=[
            pl.BlockSpec((1, _BS, _D), lambda s, b: (b, s, 0)),
            pl.BlockSpec((_BS, _D), lambda s, b: (s, 0)),
        ],
        out_specs=pl.BlockSpec((1, _BS, _D), lambda s, b: (b, s, 0)),
        out_shape=jax.ShapeDtypeStruct((_B - _SC_B, _SEQ, _D), x.dtype),
    )(x, position_embedding)


def kernel(x, position_embedding):
    sc_out = _sc_kernel(x, position_embedding)
    tc_out = _tc_kernel(x, position_embedding)
    return jnp.concatenate([tc_out, sc_out], axis=0)


# speedup vs baseline: 1.6990x; 1.6990x over previous
import functools

import jax
import jax.numpy as jnp
from jax import lax
from jax.experimental import pallas as pl
from jax.experimental.pallas import tpu as pltpu
from jax.experimental.pallas import tpu_sc as plsc

_NC, _NS = 2, 16
_NW = _NC * _NS
_D = 1024
_SEQ = 8192
_B = 4
_T = 8
_ROWS_PER_W = _SEQ // _NW
_NCHUNK = _ROWS_PER_W // _T
_NSTEP = _NCHUNK * _B
_RING = 8
_PF = 4


def _sc_body(x_hbm, pos_hbm, out_hbm, *refs):
    posb = refs[0:2]
    xb = refs[2:2 + _RING]
    ps = refs[2 + _RING:4 + _RING]
    ls = refs[4 + _RING:4 + 2 * _RING]
    ss = refs[4 + 2 * _RING:4 + 3 * _RING]
    wid = lax.axis_index("s") * _NC + lax.axis_index("c")
    base = wid * _ROWS_PER_W

    def x_slice(g):
        return (g & 3, pl.ds(base + (g >> 2) * _T, _T))

    def issue_load(g, j):
        b, rows = x_slice(g)
        pltpu.async_copy(x_hbm.at[b, rows], xb[j], ls[j])

    pltpu.async_copy(pos_hbm.at[pl.ds(base, _T)], posb[0], ps[0])
    for g0 in range(_PF):
        issue_load(g0, g0)

    @pl.loop(0, _NCHUNK, step=2)
    def _chunk_pair(cbase):
        for cc in range(2):
            c = cbase + cc
            pltpu.make_async_copy(
                pos_hbm.at[pl.ds(base + c * _T, _T)], posb[cc], ps[cc]
            ).wait()

            @pl.when(c + 1 < _NCHUNK)
            def _():
                pltpu.async_copy(
                    pos_hbm.at[pl.ds(base + (c + 1) * _T, _T)],
                    posb[1 - cc], ps[1 - cc])

            for b in range(_B):
                t = cc * _B + b
                g = cbase * _B + t
                j = t % _RING
                jn = (t + _PF) % _RING

                @pl.when(g + _PF < _NSTEP)
                def _():
                    @pl.when(g >= _RING - _PF)
                    def _():
                        bp, rp = x_slice(g + _PF - _RING)
                        pltpu.make_async_copy(
                            xb[jn], out_hbm.at[bp, rp], ss[jn]).wait()
                    issue_load(g + _PF, jn)

                bg, rg = x_slice(g)
                pltpu.make_async_copy(x_hbm.at[bg, rg], xb[j], ls[j]).wait()

                @plsc.parallel_loop(0, _T * _D, step=16, unroll=8)
                def _add(k):
                    i = k >> 10
                    col = pl.multiple_of(k & (_D - 1), 16)
                    plsc.addupdate(xb[j].at[i, pl.ds(col, 16)],
                                   posb[cc][i, pl.ds(col, 16)])

                pltpu.async_copy(xb[j], out_hbm.at[bg, rg], ss[j])

    for t in range(_RING):
        g = _NSTEP - (_RING) + t
        if g >= 0:
            bp, rp = x_slice(g)
            pltpu.make_async_copy(xb[g % _RING], out_hbm.at[bp, rp],
                                  ss[g % _RING]).wait()


_sc_kernel = functools.partial(
    pl.kernel,
    out_type=jax.ShapeDtypeStruct((_B, _SEQ, _D), jnp.float32),
    mesh=plsc.VectorSubcoreMesh(
        core_axis_name="c", subcore_axis_name="s",
        num_cores=_NC, num_subcores=_NS,
    ),
    scratch_types=(
        [pltpu.VMEM((_T, _D), jnp.float32)] * (2 + _RING)
        + [pltpu.SemaphoreType.DMA] * (2 + 2 * _RING)
    ),
)(_sc_body)


def kernel(x, position_embedding):
    return _sc_kernel(x, position_embedding)
